# Initial kernel scaffold; baseline (speedup 1.0000x reference)
import jax
import jax.numpy as jnp
from jax.experimental import pallas as pl


def kernel(x, attn_norm_scale, moe_norm_scale, Wq, bq, Wk, bk, Wv, bv, Wo, bo, gate_W, eW1, eb1, eW2, eb2, sW1, sb1, sW2, sb2):
    raise NotImplementedError("write your pallas kernel here")


# speedup vs baseline: 1.8497x; 1.8497x over previous
import functools

import jax
import jax.numpy as jnp
import numpy as np
from jax.experimental import pallas as pl
from jax.experimental.pallas import tpu as pltpu

S_, D_, H_, HD_ = 2048, 1024, 16, 64
FF_, E_, K_ = 4096, 8, 2
N_ = S_
EPS = 1e-8
T_ = 128
NT_ = N_ * K_ // T_ + E_
NPAD_ = NT_ * T_
BS_ = 256
BQ_ = 512

_f32 = jnp.float32


def _dot_t(a, b):
    return jax.lax.dot_general(a, b, (((1,), (1,)), ((), ())),
                               preferred_element_type=_f32)


def _dot(a, b):
    return jax.lax.dot_general(a, b, (((1,), (0,)), ((), ())),
                               preferred_element_type=_f32)


def _gelu(x):
    return 0.5 * x * (1.0 + jax.lax.erf(x * np.float32(1.0 / np.sqrt(2.0))))


def _rope_tables():
    inv = 1.0 / (10000.0 ** (np.arange(0, HD_, 2, dtype=np.float64) / HD_))
    inv2 = np.tile(inv, 2)
    s_idx = np.arange(S_)
    h_idx = np.arange(H_)
    pos = (s_idx[:, None] * H_ + h_idx[None, :]) % S_
    ang = pos[:, :, None].astype(np.float64) * inv2[None, None, :]
    cos = np.cos(ang).reshape(S_, D_).astype(np.float32)
    sin = np.sin(ang).reshape(S_, D_).astype(np.float32)
    return jnp.asarray(cos), jnp.asarray(sin)


_COS, _SIN = _rope_tables()

_dd = np.arange(D_) % HD_
_PERM = jnp.asarray(np.where(_dd < HD_ // 2,
                             np.arange(D_) + HD_ // 2,
                             np.arange(D_) - HD_ // 2))
_SIGN = jnp.asarray(np.where(_dd < HD_ // 2, -1.0, 1.0).astype(np.float32))


def _preattn_body(x_ref, ans_ref, cos_ref, sin_ref,
                  wq_ref, wqr_ref, wk_ref, wkr_ref, wv_ref,
                  bq_ref, bqr_ref, bk_ref, bkr_ref, bv_ref,
                  q_ref, k_ref, v_ref):
    x = x_ref[...]
    h = x * jax.lax.rsqrt(jnp.mean(x * x, axis=-1, keepdims=True) + EPS)
    h = h * ans_ref[...]
    cos = cos_ref[...]
    sin = sin_ref[...]
    q = _dot_t(h, wq_ref[...]) + bq_ref[...]
    qr = _dot_t(h, wqr_ref[...]) + bqr_ref[...]
    q_ref[...] = q * cos + qr * sin
    k = _dot_t(h, wk_ref[...]) + bk_ref[...]
    kr = _dot_t(h, wkr_ref[...]) + bkr_ref[...]
    k_ref[...] = k * cos + kr * sin
    v_ref[...] = _dot_t(h, wv_ref[...]) + bv_ref[...]


def _preattn(x, ans, Wq, Wqr, Wk, Wkr, Wv, bq, bqr, bk, bkr, bv):
    row = pl.BlockSpec((BS_, D_), lambda i: (i, 0))
    wfull = pl.BlockSpec((D_, D_), lambda i: (0, 0))
    brow = pl.BlockSpec((1, D_), lambda i: (0, 0))
    return pl.pallas_call(
        _preattn_body,
        grid=(S_ // BS_,),
        in_specs=[row, brow, row, row,
                  wfull, wfull, wfull, wfull, wfull,
                  brow, brow, brow, brow, brow],
        out_specs=[row, row, row],
        out_shape=[jax.ShapeDtypeStruct((S_, D_), _f32)] * 3,
    )(x, ans, _COS, _SIN, Wq, Wqr, Wk, Wkr, Wv,
      bq, bqr, bk, bkr, bv)


def _attn_body(q_ref, k_ref, v_ref, o_ref):
    q = q_ref[0]
    k = k_ref[0]
    s = _dot_t(q, k) * (1.0 / np.sqrt(HD_).astype(np.float32))
    m = jnp.max(s, axis=-1, keepdims=True)
    p = jnp.exp(s - m)
    p = p / jnp.sum(p, axis=-1, keepdims=True)
    o_ref[0] = _dot(p, v_ref[0])


def _attention(qh, kh, vh):
    qspec = pl.BlockSpec((1, BQ_, HD_), lambda h, i: (h, i, 0))
    kvspec = pl.BlockSpec((1, S_, HD_), lambda h, i: (h, 0, 0))
    return pl.pallas_call(
        _attn_body,
        grid=(H_, S_ // BQ_),
        in_specs=[qspec, kvspec, kvspec],
        out_specs=pl.BlockSpec((1, BQ_, HD_), lambda h, i: (h, i, 0)),
        out_shape=jax.ShapeDtypeStruct((H_, S_, HD_), _f32),
    )(qh, kh, vh)


def _postattn_body(x_ref, ao_ref, wo_ref, bo_ref, mns_ref, gw_ref,
                   x2_ref, xf_ref, lg_ref):
    attn = _dot_t(ao_ref[...], wo_ref[...]) + bo_ref[...]
    x2 = x_ref[...] + attn
    x2_ref[...] = x2
    h = x2 * jax.lax.rsqrt(jnp.mean(x2 * x2, axis=-1, keepdims=True) + EPS)
    h = h * mns_ref[...]
    xf_ref[...] = h
    lg_ref[...] = _dot_t(h, gw_ref[...])


def _postattn(x, ao, Wo, bo, mns, gw_pad):
    row = pl.BlockSpec((BS_, D_), lambda i: (i, 0))
    wfull = pl.BlockSpec((D_, D_), lambda i: (0, 0))
    brow = pl.BlockSpec((1, D_), lambda i: (0, 0))
    gspec = pl.BlockSpec((128, D_), lambda i: (0, 0))
    lspec = pl.BlockSpec((BS_, 128), lambda i: (i, 0))
    return pl.pallas_call(
        _postattn_body,
        grid=(S_ // BS_,),
        in_specs=[row, row, wfull, brow, brow, gspec],
        out_specs=[row, row, lspec],
        out_shape=[jax.ShapeDtypeStruct((S_, D_), _f32),
                   jax.ShapeDtypeStruct((S_, D_), _f32),
                   jax.ShapeDtypeStruct((S_, 128), _f32)],
    )(x, ao, Wo, bo, mns, gw_pad)


def _ffn1_body(xf_ref, w1_ref, b1_ref, h1_ref):
    h1_ref[...] = _gelu(_dot_t(xf_ref[...], w1_ref[...]) + b1_ref[...])


def _ffn2_body(h1_ref, x2_ref, w2_ref, b2_ref, out_ref):
    out_ref[...] = x2_ref[...] + _dot_t(h1_ref[...], w2_ref[...]) + b2_ref[...]


def _sharedffn(xf, x2, sW1, sb1, sW2, sb2):
    row = pl.BlockSpec((BS_, D_), lambda i: (i, 0))
    ffrow = pl.BlockSpec((BS_, FF_), lambda i: (i, 0))
    h1 = pl.pallas_call(
        _ffn1_body,
        grid=(S_ // BS_,),
        in_specs=[row, pl.BlockSpec((FF_, D_), lambda i: (0, 0)),
                  pl.BlockSpec((1, FF_), lambda i: (0, 0))],
        out_specs=ffrow,
        out_shape=jax.ShapeDtypeStruct((S_, FF_), _f32),
    )(xf, sW1, sb1)
    return pl.pallas_call(
        _ffn2_body,
        grid=(S_ // BS_,),
        in_specs=[ffrow, row, pl.BlockSpec((D_, FF_), lambda i: (0, 0)),
                  pl.BlockSpec((1, D_), lambda i: (0, 0))],
        out_specs=row,
        out_shape=jax.ShapeDtypeStruct((S_, D_), _f32),
    )(h1, x2, sW2, sb2)


def _moe1_body(te_ref, px_ref, w1_ref, b1_ref, h1_ref):
    del te_ref
    h1_ref[...] = _gelu(_dot_t(px_ref[...], w1_ref[0]) + b1_ref[0])


def _moe2_body(te_ref, h1_ref, w2_ref, b2_ref, wpad_ref, out_ref):
    del te_ref
    acc = _dot_t(h1_ref[...], w2_ref[0]) + b2_ref[0]
    out_ref[...] = acc * wpad_ref[...][:, 0:1]


def _moe_ffn(px, eW1, eb1, eW2, eb2, wpad2d, tile_expert):
    trow = pl.BlockSpec((T_, D_), lambda i, te: (i, 0))
    tff = pl.BlockSpec((T_, FF_), lambda i, te: (i, 0))
    h1 = pl.pallas_call(
        _moe1_body,
        grid_spec=pltpu.PrefetchScalarGridSpec(
            num_scalar_prefetch=1,
            grid=(NT_,),
            in_specs=[
                trow,
                pl.BlockSpec((1, FF_, D_), lambda i, te: (te[i], 0, 0)),
                pl.BlockSpec((1, 1, FF_), lambda i, te: (te[i], 0, 0)),
            ],
            out_specs=tff,
        ),
        out_shape=jax.ShapeDtypeStruct((NPAD_, FF_), _f32),
    )(tile_expert, px, eW1, eb1)
    return pl.pallas_call(
        _moe2_body,
        grid_spec=pltpu.PrefetchScalarGridSpec(
            num_scalar_prefetch=1,
            grid=(NT_,),
            in_specs=[
                tff,
                pl.BlockSpec((1, D_, FF_), lambda i, te: (te[i], 0, 0)),
                pl.BlockSpec((1, 1, D_), lambda i, te: (te[i], 0, 0)),
                pl.BlockSpec((T_, 128), lambda i, te: (i, 0)),
            ],
            out_specs=trow,
        ),
        out_shape=jax.ShapeDtypeStruct((NPAD_, D_), _f32),
    )(tile_expert, h1, eW2, eb2, wpad2d)


def _final_body(sh_ref, g0_ref, g1_ref, out_ref):
    out_ref[...] = sh_ref[...] + g0_ref[...] + g1_ref[...]


def _final(sh, g0, g1):
    row = pl.BlockSpec((BS_, D_), lambda i: (i, 0))
    return pl.pallas_call(
        _final_body,
        grid=(S_ // BS_,),
        in_specs=[row, row, row],
        out_specs=row,
        out_shape=jax.ShapeDtypeStruct((S_, D_), _f32),
    )(sh, g0, g1)


def kernel(x, attn_norm_scale, moe_norm_scale, Wq, bq, Wk, bk, Wv, bv, Wo, bo,
           gate_W, eW1, eb1, eW2, eb2, sW1, sb1, sW2, sb2):
    xf0 = x.reshape(S_, D_)
    ans = attn_norm_scale.reshape(1, D_)
    mns = moe_norm_scale.reshape(1, D_)

    Wqr = _SIGN[:, None] * Wq[_PERM]
    bqr = _SIGN * bq[_PERM]
    Wkr = _SIGN[:, None] * Wk[_PERM]
    bkr = _SIGN * bk[_PERM]

    q, k, v = _preattn(xf0, ans, Wq, Wqr, Wk, Wkr, Wv,
                       bq.reshape(1, D_), bqr.reshape(1, D_),
                       bk.reshape(1, D_), bkr.reshape(1, D_),
                       bv.reshape(1, D_))
    qh = q.reshape(S_, H_, HD_).transpose(1, 0, 2)
    kh = k.reshape(S_, H_, HD_).transpose(1, 0, 2)
    vh = v.reshape(S_, H_, HD_).transpose(1, 0, 2)
    ao = _attention(qh, kh, vh).transpose(1, 0, 2).reshape(S_, D_)

    gw_pad = jnp.zeros((128, D_), _f32).at[:E_].set(gate_W)
    x2, xf, lg = _postattn(xf0, ao, Wo, bo.reshape(1, D_), mns, gw_pad)

    logits = lg[:, :E_]
    topv, topi = jax.lax.top_k(logits, K_)
    probs = jax.nn.softmax(topv, axis=-1)
    expert_mask = jax.nn.one_hot(topi, E_, dtype=_f32)

    onehot_p = expert_mask.reshape(-1, E_)
    counts = jnp.sum(onehot_p, axis=0).astype(jnp.int32)
    rank = jnp.sum((jnp.cumsum(onehot_p, axis=0) - onehot_p) * onehot_p,
                   axis=1).astype(jnp.int32)
    flat_e = topi.reshape(-1)
    padded_cnt = ((counts + T_ - 1) // T_) * T_
    pend = jnp.cumsum(padded_cnt)
    pstart = pend - padded_cnt
    slot = pstart[flat_e] + rank
    token_of_pair = jnp.arange(N_ * K_, dtype=jnp.int32) // K_
    src_token = jnp.zeros((NPAD_,), jnp.int32).at[slot].set(token_of_pair)
    w_pad = jnp.zeros((NPAD_,), _f32).at[slot].set(probs.reshape(-1))
    slot2 = slot.reshape(N_, K_)
    tile_expert = jnp.minimum(
        jnp.searchsorted(pend, jnp.arange(NT_, dtype=jnp.int32) * T_,
                         side='right'),
        E_ - 1).astype(jnp.int32)

    px = xf[src_token]
    wpad2d = jnp.broadcast_to(w_pad[:, None], (NPAD_, 128))
    pout = _moe_ffn(px, eW1, eb1.reshape(E_, 1, FF_), eW2,
                    eb2.reshape(E_, 1, D_), wpad2d, tile_expert)
    g0 = pout[slot2[:, 0]]
    g1 = pout[slot2[:, 1]]

    sh = _sharedffn(xf, x2, sW1, sb1.reshape(1, FF_), sW2, sb2.reshape(1, D_))
    out = _final(sh, g0, g1)
    return out.reshape(1, S_, D_), expert_mask
